# baseline (speedup 1.0000x reference)
import functools

import jax
import jax.numpy as jnp
import numpy as np
from jax import lax
from jax.experimental import pallas as pl
from jax.experimental.pallas import tpu as pltpu
from jax.experimental.pallas import tpu_sc as plsc

EMBED_DIM = 512
NUM_SURV = 10
N_NODES = 10000
BATCH = 8192
_ROWS_PER_BLK = 1000


_BLK = 1024
_N_PAD = 10240


def _pq_body(w_ref, e_ref, p_ref, q_ref):
    i = pl.program_id(0)
    e = e_ref[...]
    wu = w_ref[:, :EMBED_DIM]
    wv = w_ref[:, EMBED_DIM:]
    dn = (((1,), (1,)), ((), ()))
    pt = lax.dot_general(wu, e, dn, preferred_element_type=jnp.float32)
    qt = lax.dot_general(wv, e, dn, preferred_element_type=jnp.float32)
    sl = pl.ds(i * _BLK, _BLK)
    p_ref[sl] = pt.reshape(_BLK)
    q_ref[sl] = qt.reshape(_BLK)


def _compute_pq(all_embeddings, w_omega):
    p, q = pl.pallas_call(
        _pq_body,
        grid=(_N_PAD // _BLK,),
        in_specs=[
            pl.BlockSpec((1, 2 * EMBED_DIM), lambda i: (0, 0)),
            pl.BlockSpec((_BLK, EMBED_DIM), lambda i: (i, 0)),
        ],
        out_specs=[
            pl.BlockSpec((_N_PAD,), lambda i: (0,)),
            pl.BlockSpec((_N_PAD,), lambda i: (0,)),
        ],
        out_shape=[
            jax.ShapeDtypeStruct((_N_PAD,), jnp.float32),
            jax.ShapeDtypeStruct((_N_PAD,), jnp.float32),
        ],
    )(w_omega, all_embeddings)
    return p, q


def _sc_mesh():
    mesh = plsc.VectorSubcoreMesh(core_axis_name="c", subcore_axis_name="s")
    nw = mesh.num_cores * mesh.num_subcores
    return mesh, nw, BATCH // nw, (BATCH * NUM_SURV) // nw


def _sc_compose(assoc, src, pos_dst, neg_dst_surv, neg_src_surv):
    mesh, nw, nb, ns = _sc_mesh()
    num_cores = mesh.num_cores

    @functools.partial(
        pl.kernel,
        out_type=[
            jax.ShapeDtypeStruct((BATCH,), jnp.int32),
            jax.ShapeDtypeStruct((BATCH,), jnp.int32),
            jax.ShapeDtypeStruct((BATCH * NUM_SURV,), jnp.int32),
            jax.ShapeDtypeStruct((BATCH * NUM_SURV,), jnp.int32),
        ],
        mesh=mesh,
        compiler_params=pltpu.CompilerParams(needs_layout_passes=False),
        scratch_types=[
            pltpu.VMEM((N_NODES,), jnp.int32),
            pltpu.VMEM((nb,), jnp.int32),
            pltpu.VMEM((nb,), jnp.int32),
            pltpu.VMEM((ns,), jnp.int32),
            pltpu.VMEM((ns,), jnp.int32),
            pltpu.VMEM((nb,), jnp.int32),
            pltpu.VMEM((nb,), jnp.int32),
            pltpu.VMEM((ns,), jnp.int32),
            pltpu.VMEM((ns,), jnp.int32),
            pltpu.SemaphoreType.DMA,
            pltpu.SemaphoreType.DMA,
        ],
    )
    def k(assoc_h, src_h, pos_h, negd_h, negs_h,
          oisrc_h, oidst_h, oind_h, oins_h,
          assoc_v, src_v, pos_v, negd_v, negs_v,
          isrc_v, idst_v, ind_v, ins_v, sem_in, sem_out):
        wid = lax.axis_index("s") * num_cores + lax.axis_index("c")
        copies = [
            pltpu.async_copy(src_h.at[pl.ds(wid * nb, nb)], src_v, sem_in),
            pltpu.async_copy(pos_h.at[pl.ds(wid * nb, nb)], pos_v, sem_in),
            pltpu.async_copy(negd_h.at[pl.ds(wid * ns, ns)], negd_v, sem_in),
            pltpu.async_copy(negs_h.at[pl.ds(wid * ns, ns)], negs_v, sem_in),
            pltpu.async_copy(assoc_h, assoc_v, sem_in),
        ]
        for c in copies:
            c.wait()

        def lam_body(t, carry):
            for u in range(2):
                sl = pl.ds(t * 32 + u * 16, 16)
                isrc_v[sl] = plsc.load_gather(assoc_v, [src_v[sl]])
                idst_v[sl] = plsc.load_gather(assoc_v, [pos_v[sl]])
            return carry

        lax.fori_loop(0, nb // 32, lam_body, 0)
        o1 = pltpu.async_copy(isrc_v, oisrc_h.at[pl.ds(wid * nb, nb)], sem_out)
        o2 = pltpu.async_copy(idst_v, oidst_h.at[pl.ds(wid * nb, nb)], sem_out)

        def surv_body(t, carry):
            for u in range(4):
                sl = pl.ds(t * 64 + u * 16, 16)
                ind_v[sl] = plsc.load_gather(assoc_v, [negd_v[sl]])
                ins_v[sl] = plsc.load_gather(assoc_v, [negs_v[sl]])
            return carry

        lax.fori_loop(0, ns // 64, surv_body, 0)
        o3 = pltpu.async_copy(ind_v, oind_h.at[pl.ds(wid * ns, ns)], sem_out)
        o4 = pltpu.async_copy(ins_v, oins_h.at[pl.ds(wid * ns, ns)], sem_out)
        o1.wait()
        o2.wait()
        o3.wait()
        o4.wait()

    return k(assoc, src, pos_dst, neg_dst_surv, neg_src_surv)


def _sc_logits(p, q, isrc, idst, ind, ins):
    mesh, nw, nb, ns = _sc_mesh()
    num_cores = mesh.num_cores
    rep_idx = jnp.asarray(np.arange(ns, dtype=np.int32) // NUM_SURV)

    @functools.partial(
        pl.kernel,
        out_type=[
            jax.ShapeDtypeStruct((BATCH,), jnp.float32),
            jax.ShapeDtypeStruct((BATCH * NUM_SURV,), jnp.float32),
            jax.ShapeDtypeStruct((BATCH * NUM_SURV,), jnp.float32),
        ],
        mesh=mesh,
        compiler_params=pltpu.CompilerParams(needs_layout_passes=False),
        scratch_types=[
            pltpu.VMEM((_N_PAD,), jnp.float32),
            pltpu.VMEM((_N_PAD,), jnp.float32),
            pltpu.VMEM((nb,), jnp.int32),
            pltpu.VMEM((nb,), jnp.int32),
            pltpu.VMEM((ns,), jnp.int32),
            pltpu.VMEM((ns,), jnp.int32),
            pltpu.VMEM((ns,), jnp.int32),
            pltpu.VMEM((nb,), jnp.float32),
            pltpu.VMEM((nb,), jnp.float32),
            pltpu.VMEM((nb,), jnp.float32),
            pltpu.VMEM((ns,), jnp.float32),
            pltpu.VMEM((ns,), jnp.float32),
            pltpu.SemaphoreType.DMA,
            pltpu.SemaphoreType.DMA,
        ],
    )
    def k(p_h, q_h, isrc_h, idst_h, ind_h, ins_h, rep_h,
          olam_h, osu_h, osv_h,
          p_v, q_v, isrc_v, idst_v, ind_v, ins_v, rep_v,
          psrc_v, qdst_v, lam_v, su_v, sv_v, sem_in, sem_out):
        wid = lax.axis_index("s") * num_cores + lax.axis_index("c")
        lam_copies = [
            pltpu.async_copy(isrc_h.at[pl.ds(wid * nb, nb)], isrc_v, sem_in),
            pltpu.async_copy(idst_h.at[pl.ds(wid * nb, nb)], idst_v, sem_in),
            pltpu.async_copy(p_h, p_v, sem_in),
            pltpu.async_copy(q_h, q_v, sem_in),
        ]
        surv_copies = [
            pltpu.async_copy(ind_h.at[pl.ds(wid * ns, ns)], ind_v, sem_in),
            pltpu.async_copy(ins_h.at[pl.ds(wid * ns, ns)], ins_v, sem_in),
            pltpu.async_copy(rep_h, rep_v, sem_in),
        ]
        for c in lam_copies:
            c.wait()

        def lam_body(t, carry):
            for u in range(2):
                sl = pl.ds(t * 32 + u * 16, 16)
                pe = plsc.load_gather(p_v, [isrc_v[sl]])
                qe = plsc.load_gather(q_v, [idst_v[sl]])
                psrc_v[sl] = pe
                qdst_v[sl] = qe
                lam_v[sl] = pe + qe
            return carry

        lax.fori_loop(0, nb // 32, lam_body, 0)
        out_lam = pltpu.async_copy(lam_v, olam_h.at[pl.ds(wid * nb, nb)], sem_out)
        for c in surv_copies:
            c.wait()

        def surv_body(t, carry):
            for u in range(4):
                sl = pl.ds(t * 64 + u * 16, 16)
                ri = rep_v[sl]
                pe = plsc.load_gather(psrc_v, [ri])
                qe = plsc.load_gather(q_v, [ind_v[sl]])
                su_v[sl] = pe + qe
                pe2 = plsc.load_gather(p_v, [ins_v[sl]])
                qe2 = plsc.load_gather(qdst_v, [ri])
                sv_v[sl] = pe2 + qe2
            return carry

        lax.fori_loop(0, ns // 64, surv_body, 0)

        out_su = pltpu.async_copy(su_v, osu_h.at[pl.ds(wid * ns, ns)], sem_out)
        out_sv = pltpu.async_copy(sv_v, osv_h.at[pl.ds(wid * ns, ns)], sem_out)
        out_lam.wait()
        out_su.wait()
        out_sv.wait()

    return k(p, q, isrc, idst, ind, ins, rep_idx)


_LOSS_GRID = 5
_SU_ROWS = BATCH * NUM_SURV // 128 // _LOSS_GRID


def _loss_body(b_ref, psi_ref, slam_ref, ssu_ref, ssv_ref, o1, o2, o3, acc):
    i = pl.program_id(0)
    b = b_ref[0]
    psi = psi_ref[0]
    pe = psi + 1e-7
    gu = (ssu_ref[...] + b) / pe
    su_s = jnp.sum(jnp.log(1.0 + jnp.exp(-gu)) + gu)
    gv = (ssv_ref[...] + b) / pe
    sv_s = jnp.sum(jnp.log(1.0 + jnp.exp(-gv)) + gv)

    @pl.when(i == 0)
    def _():
        gl = (slam_ref[...] + b) / pe
        lam = psi * (jnp.log(1.0 + jnp.exp(-gl)) + gl)
        acc[0] = -jnp.sum(jnp.log(lam + 1e-10))
        acc[1] = su_s
        acc[2] = sv_s

    @pl.when(i > 0)
    def _():
        acc[1] = acc[1] + su_s
        acc[2] = acc[2] + sv_s

    @pl.when(i == _LOSS_GRID - 1)
    def _():
        o1[0] = acc[0] / BATCH
        o2[0] = psi * acc[1] / NUM_SURV / BATCH
        o3[0] = psi * acc[2] / NUM_SURV / BATCH


def _losses(s_lam, s_su, s_sv, b_omega, psi):
    o1, o2, o3 = pl.pallas_call(
        _loss_body,
        grid=(_LOSS_GRID,),
        in_specs=[
            pl.BlockSpec(memory_space=pltpu.SMEM),
            pl.BlockSpec(memory_space=pltpu.SMEM),
            pl.BlockSpec((BATCH // 128, 128), lambda i: (0, 0)),
            pl.BlockSpec((_SU_ROWS, 128), lambda i: (i, 0)),
            pl.BlockSpec((_SU_ROWS, 128), lambda i: (i, 0)),
        ],
        out_specs=[
            pl.BlockSpec(memory_space=pltpu.SMEM),
            pl.BlockSpec(memory_space=pltpu.SMEM),
            pl.BlockSpec(memory_space=pltpu.SMEM),
        ],
        out_shape=[jax.ShapeDtypeStruct((1,), jnp.float32)] * 3,
        scratch_shapes=[pltpu.SMEM((3,), jnp.float32)],
    )(
        b_omega, psi,
        s_lam.reshape(BATCH // 128, 128),
        s_su.reshape(BATCH * NUM_SURV // 128, 128),
        s_sv.reshape(BATCH * NUM_SURV // 128, 128),
    )
    return o1[0], o2[0], o3[0]


def kernel(all_embeddings, assoc, src, pos_dst, neg_dst_surv, neg_src_surv,
           W_omega, b_omega, psi):
    assoc_i = assoc.astype(jnp.int32)
    isrc, idst, ind, ins = _sc_compose(
        assoc_i, src, pos_dst, neg_dst_surv, neg_src_surv)
    p, q = _compute_pq(all_embeddings, W_omega)
    s_lam, s_su, s_sv = _sc_logits(p, q, isrc, idst, ind, ins)
    return _losses(s_lam, s_su, s_sv, b_omega, psi)


# speedup vs baseline: 1.0444x; 1.0444x over previous
import functools

import jax
import jax.numpy as jnp
import numpy as np
from jax import lax
from jax.experimental import pallas as pl
from jax.experimental.pallas import tpu as pltpu
from jax.experimental.pallas import tpu_sc as plsc

EMBED_DIM = 512
NUM_SURV = 10
N_NODES = 10000
BATCH = 8192
_ROWS_PER_BLK = 1000


_BLK = 1024
_N_PAD = 10240


def _pq_body(w_ref, e_ref, p_hbm, q_hbm, p_acc, q_acc, sem):
    i = pl.program_id(0)
    e = e_ref[...]
    wu = w_ref[:, :EMBED_DIM]
    wv = w_ref[:, EMBED_DIM:]
    dn = (((1,), (1,)), ((), ()))
    pt = lax.dot_general(wu, e, dn, preferred_element_type=jnp.float32)
    qt = lax.dot_general(wv, e, dn, preferred_element_type=jnp.float32)
    sl = pl.ds(i * _BLK, _BLK)
    p_acc[sl] = pt.reshape(_BLK)
    q_acc[sl] = qt.reshape(_BLK)

    @pl.when(i == _N_PAD // _BLK - 1)
    def _():
        cp = pltpu.make_async_copy(p_acc, p_hbm, sem)
        cp.start()
        cq = pltpu.make_async_copy(q_acc, q_hbm, sem)
        cq.start()
        cp.wait()
        cq.wait()


def _compute_pq(all_embeddings, w_omega):
    p, q = pl.pallas_call(
        _pq_body,
        grid=(_N_PAD // _BLK,),
        in_specs=[
            pl.BlockSpec((1, 2 * EMBED_DIM), lambda i: (0, 0)),
            pl.BlockSpec((_BLK, EMBED_DIM), lambda i: (i, 0)),
        ],
        out_specs=[
            pl.BlockSpec(memory_space=pl.ANY),
            pl.BlockSpec(memory_space=pl.ANY),
        ],
        out_shape=[
            jax.ShapeDtypeStruct((_N_PAD,), jnp.float32),
            jax.ShapeDtypeStruct((_N_PAD,), jnp.float32),
        ],
        scratch_shapes=[
            pltpu.VMEM((_N_PAD,), jnp.float32),
            pltpu.VMEM((_N_PAD,), jnp.float32),
            pltpu.SemaphoreType.DMA,
        ],
    )(w_omega, all_embeddings)
    return p, q


def _sc_mesh():
    mesh = plsc.VectorSubcoreMesh(core_axis_name="c", subcore_axis_name="s")
    nw = mesh.num_cores * mesh.num_subcores
    return mesh, nw, BATCH // nw, (BATCH * NUM_SURV) // nw


def _sc_compose(assoc, src, pos_dst, neg_dst_surv, neg_src_surv):
    mesh, nw, nb, ns = _sc_mesh()
    num_cores = mesh.num_cores

    @functools.partial(
        pl.kernel,
        out_type=[
            jax.ShapeDtypeStruct((BATCH,), jnp.int32),
            jax.ShapeDtypeStruct((BATCH,), jnp.int32),
            jax.ShapeDtypeStruct((BATCH * NUM_SURV,), jnp.int32),
            jax.ShapeDtypeStruct((BATCH * NUM_SURV,), jnp.int32),
        ],
        mesh=mesh,
        compiler_params=pltpu.CompilerParams(needs_layout_passes=False),
        scratch_types=[
            pltpu.VMEM((N_NODES,), jnp.int32),
            pltpu.VMEM((nb,), jnp.int32),
            pltpu.VMEM((nb,), jnp.int32),
            pltpu.VMEM((ns,), jnp.int32),
            pltpu.VMEM((ns,), jnp.int32),
            pltpu.VMEM((nb,), jnp.int32),
            pltpu.VMEM((nb,), jnp.int32),
            pltpu.VMEM((ns,), jnp.int32),
            pltpu.VMEM((ns,), jnp.int32),
            pltpu.SemaphoreType.DMA,
            pltpu.SemaphoreType.DMA,
        ],
    )
    def k(assoc_h, src_h, pos_h, negd_h, negs_h,
          oisrc_h, oidst_h, oind_h, oins_h,
          assoc_v, src_v, pos_v, negd_v, negs_v,
          isrc_v, idst_v, ind_v, ins_v, sem_in, sem_out):
        wid = lax.axis_index("s") * num_cores + lax.axis_index("c")
        copies = [
            pltpu.async_copy(src_h.at[pl.ds(wid * nb, nb)], src_v, sem_in),
            pltpu.async_copy(pos_h.at[pl.ds(wid * nb, nb)], pos_v, sem_in),
            pltpu.async_copy(negd_h.at[pl.ds(wid * ns, ns)], negd_v, sem_in),
            pltpu.async_copy(negs_h.at[pl.ds(wid * ns, ns)], negs_v, sem_in),
            pltpu.async_copy(assoc_h, assoc_v, sem_in),
        ]
        for c in copies:
            c.wait()

        def lam_body(t, carry):
            for u in range(2):
                sl = pl.ds(t * 32 + u * 16, 16)
                isrc_v[sl] = plsc.load_gather(assoc_v, [src_v[sl]])
                idst_v[sl] = plsc.load_gather(assoc_v, [pos_v[sl]])
            return carry

        lax.fori_loop(0, nb // 32, lam_body, 0)
        o1 = pltpu.async_copy(isrc_v, oisrc_h.at[pl.ds(wid * nb, nb)], sem_out)
        o2 = pltpu.async_copy(idst_v, oidst_h.at[pl.ds(wid * nb, nb)], sem_out)

        def surv_body(t, carry):
            for u in range(4):
                sl = pl.ds(t * 64 + u * 16, 16)
                ind_v[sl] = plsc.load_gather(assoc_v, [negd_v[sl]])
                ins_v[sl] = plsc.load_gather(assoc_v, [negs_v[sl]])
            return carry

        lax.fori_loop(0, ns // 64, surv_body, 0)
        o3 = pltpu.async_copy(ind_v, oind_h.at[pl.ds(wid * ns, ns)], sem_out)
        o4 = pltpu.async_copy(ins_v, oins_h.at[pl.ds(wid * ns, ns)], sem_out)
        o1.wait()
        o2.wait()
        o3.wait()
        o4.wait()

    return k(assoc, src, pos_dst, neg_dst_surv, neg_src_surv)


def _sc_logits(p, q, isrc, idst, ind, ins):
    mesh, nw, nb, ns = _sc_mesh()
    num_cores = mesh.num_cores
    rep_idx = jnp.asarray(np.arange(ns, dtype=np.int32) // NUM_SURV)

    @functools.partial(
        pl.kernel,
        out_type=[
            jax.ShapeDtypeStruct((BATCH,), jnp.float32),
            jax.ShapeDtypeStruct((BATCH * NUM_SURV,), jnp.float32),
            jax.ShapeDtypeStruct((BATCH * NUM_SURV,), jnp.float32),
        ],
        mesh=mesh,
        compiler_params=pltpu.CompilerParams(needs_layout_passes=False),
        scratch_types=[
            pltpu.VMEM((_N_PAD,), jnp.float32),
            pltpu.VMEM((_N_PAD,), jnp.float32),
            pltpu.VMEM((nb,), jnp.int32),
            pltpu.VMEM((nb,), jnp.int32),
            pltpu.VMEM((ns,), jnp.int32),
            pltpu.VMEM((ns,), jnp.int32),
            pltpu.VMEM((ns,), jnp.int32),
            pltpu.VMEM((nb,), jnp.float32),
            pltpu.VMEM((nb,), jnp.float32),
            pltpu.VMEM((nb,), jnp.float32),
            pltpu.VMEM((ns,), jnp.float32),
            pltpu.VMEM((ns,), jnp.float32),
            pltpu.SemaphoreType.DMA,
            pltpu.SemaphoreType.DMA,
            pltpu.SemaphoreType.DMA,
        ],
    )
    def k(p_h, q_h, isrc_h, idst_h, ind_h, ins_h, rep_h,
          olam_h, osu_h, osv_h,
          p_v, q_v, isrc_v, idst_v, ind_v, ins_v, rep_v,
          psrc_v, qdst_v, lam_v, su_v, sv_v, sem_lam, sem_surv, sem_out):
        wid = lax.axis_index("s") * num_cores + lax.axis_index("c")
        lam_copies = [
            pltpu.async_copy(isrc_h.at[pl.ds(wid * nb, nb)], isrc_v, sem_lam),
            pltpu.async_copy(idst_h.at[pl.ds(wid * nb, nb)], idst_v, sem_lam),
            pltpu.async_copy(p_h, p_v, sem_lam),
            pltpu.async_copy(q_h, q_v, sem_lam),
        ]
        surv_copies = [
            pltpu.async_copy(ind_h.at[pl.ds(wid * ns, ns)], ind_v, sem_surv),
            pltpu.async_copy(ins_h.at[pl.ds(wid * ns, ns)], ins_v, sem_surv),
            pltpu.async_copy(rep_h, rep_v, sem_surv),
        ]
        for c in lam_copies:
            c.wait()

        def lam_body(t, carry):
            for u in range(2):
                sl = pl.ds(t * 32 + u * 16, 16)
                pe = plsc.load_gather(p_v, [isrc_v[sl]])
                qe = plsc.load_gather(q_v, [idst_v[sl]])
                psrc_v[sl] = pe
                qdst_v[sl] = qe
                lam_v[sl] = pe + qe
            return carry

        lax.fori_loop(0, nb // 32, lam_body, 0)
        out_lam = pltpu.async_copy(lam_v, olam_h.at[pl.ds(wid * nb, nb)], sem_out)
        for c in surv_copies:
            c.wait()

        def surv_body(t, carry):
            for u in range(4):
                sl = pl.ds(t * 64 + u * 16, 16)
                ri = rep_v[sl]
                pe = plsc.load_gather(psrc_v, [ri])
                qe = plsc.load_gather(q_v, [ind_v[sl]])
                su_v[sl] = pe + qe
                pe2 = plsc.load_gather(p_v, [ins_v[sl]])
                qe2 = plsc.load_gather(qdst_v, [ri])
                sv_v[sl] = pe2 + qe2
            return carry

        lax.fori_loop(0, ns // 64, surv_body, 0)

        out_su = pltpu.async_copy(su_v, osu_h.at[pl.ds(wid * ns, ns)], sem_out)
        out_sv = pltpu.async_copy(sv_v, osv_h.at[pl.ds(wid * ns, ns)], sem_out)
        out_lam.wait()
        out_su.wait()
        out_sv.wait()

    return k(p, q, isrc, idst, ind, ins, rep_idx)


def _loss_body(b_ref, psi_ref, slam_ref, ssu_ref, ssv_ref, o1, o2, o3):
    b = b_ref[0]
    psi = psi_ref[0]
    pe = psi + 1e-7
    gl = (slam_ref[...] + b) / pe
    lam = psi * (jnp.log(1.0 + jnp.exp(-gl)) + gl)
    o1[0] = -jnp.sum(jnp.log(lam + 1e-10)) / BATCH
    gu = (ssu_ref[...] + b) / pe
    o2[0] = psi * jnp.sum(jnp.log(1.0 + jnp.exp(-gu)) + gu) / NUM_SURV / BATCH
    gv = (ssv_ref[...] + b) / pe
    o3[0] = psi * jnp.sum(jnp.log(1.0 + jnp.exp(-gv)) + gv) / NUM_SURV / BATCH


def _losses(s_lam, s_su, s_sv, b_omega, psi):
    o1, o2, o3 = pl.pallas_call(
        _loss_body,
        in_specs=[
            pl.BlockSpec(memory_space=pltpu.SMEM),
            pl.BlockSpec(memory_space=pltpu.SMEM),
            pl.BlockSpec((BATCH // 128, 128), lambda: (0, 0)),
            pl.BlockSpec((BATCH * NUM_SURV // 128, 128), lambda: (0, 0)),
            pl.BlockSpec((BATCH * NUM_SURV // 128, 128), lambda: (0, 0)),
        ],
        out_specs=[
            pl.BlockSpec(memory_space=pltpu.SMEM),
            pl.BlockSpec(memory_space=pltpu.SMEM),
            pl.BlockSpec(memory_space=pltpu.SMEM),
        ],
        out_shape=[jax.ShapeDtypeStruct((1,), jnp.float32)] * 3,
    )(
        b_omega, psi,
        s_lam.reshape(BATCH // 128, 128),
        s_su.reshape(BATCH * NUM_SURV // 128, 128),
        s_sv.reshape(BATCH * NUM_SURV // 128, 128),
    )
    return o1[0], o2[0], o3[0]


def kernel(all_embeddings, assoc, src, pos_dst, neg_dst_surv, neg_src_surv,
           W_omega, b_omega, psi):
    assoc_i = assoc.astype(jnp.int32)
    isrc, idst, ind, ins = _sc_compose(
        assoc_i, src, pos_dst, neg_dst_surv, neg_src_surv)
    p, q = _compute_pq(all_embeddings, W_omega)
    s_lam, s_su, s_sv = _sc_logits(p, q, isrc, idst, ind, ins)
    return _losses(s_lam, s_su, s_sv, b_omega, psi)


# speedup vs baseline: 1.0683x; 1.0229x over previous
import functools

import jax
import jax.numpy as jnp
from jax import lax
from jax.experimental import pallas as pl
from jax.experimental.pallas import tpu as pltpu
from jax.experimental.pallas import tpu_sc as plsc

EMBED_DIM = 512
NUM_SURV = 10
N_NODES = 10000
BATCH = 8192
_ROWS_PER_BLK = 1000


_N_PAD = 10240
_MBLK = 512
_MNFULL = N_NODES // _MBLK
_MTAIL = N_NODES - _MNFULL * _MBLK
_NBUF = 4


def _pq_body(w_ref, e_hbm, p_hbm, q_hbm, p_acc, q_acc, ebuf, sems, osem):
    wu = w_ref[:, :EMBED_DIM]
    wv = w_ref[:, EMBED_DIM:]
    dn = (((1,), (1,)), ((), ()))
    nblk = _MNFULL + 1

    def _copy(t):
        rows = _MBLK if t < _MNFULL else _MTAIL
        return pltpu.make_async_copy(
            e_hbm.at[pl.ds(t * _MBLK, rows), :],
            ebuf.at[t % _NBUF, pl.ds(0, rows)],
            sems.at[t % _NBUF],
        )

    for s in range(_NBUF):
        _copy(s).start()
    for t in range(nblk):
        rows = _MBLK if t < _MNFULL else _MTAIL
        _copy(t).wait()
        e = ebuf[t % _NBUF, pl.ds(0, rows)]
        pt = lax.dot_general(wu, e, dn, preferred_element_type=jnp.float32)
        qt = lax.dot_general(wv, e, dn, preferred_element_type=jnp.float32)
        sl = pl.ds(t * _MBLK, rows)
        p_acc[sl] = pt.reshape(rows)
        q_acc[sl] = qt.reshape(rows)
        if t + _NBUF < nblk:
            _copy(t + _NBUF).start()
    cp = pltpu.make_async_copy(p_acc, p_hbm, osem)
    cp.start()
    cq = pltpu.make_async_copy(q_acc, q_hbm, osem)
    cq.start()
    cp.wait()
    cq.wait()


def _compute_pq(all_embeddings, w_omega):
    p, q = pl.pallas_call(
        _pq_body,
        in_specs=[
            pl.BlockSpec((1, 2 * EMBED_DIM), lambda: (0, 0)),
            pl.BlockSpec(memory_space=pl.ANY),
        ],
        out_specs=[
            pl.BlockSpec(memory_space=pl.ANY),
            pl.BlockSpec(memory_space=pl.ANY),
        ],
        out_shape=[
            jax.ShapeDtypeStruct((_N_PAD,), jnp.float32),
            jax.ShapeDtypeStruct((_N_PAD,), jnp.float32),
        ],
        scratch_shapes=[
            pltpu.VMEM((_N_PAD,), jnp.float32),
            pltpu.VMEM((_N_PAD,), jnp.float32),
            pltpu.VMEM((_NBUF, _MBLK, EMBED_DIM), jnp.float32),
            pltpu.SemaphoreType.DMA((_NBUF,)),
            pltpu.SemaphoreType.DMA,
        ],
    )(w_omega, all_embeddings)
    return p, q


def _sc_mesh():
    mesh = plsc.VectorSubcoreMesh(core_axis_name="c", subcore_axis_name="s")
    nw = mesh.num_cores * mesh.num_subcores
    return mesh, nw, BATCH // nw, (BATCH * NUM_SURV) // nw


def _sc_compose(assoc, src, pos_dst, neg_dst_surv, neg_src_surv):
    mesh, nw, nb, ns = _sc_mesh()
    num_cores = mesh.num_cores

    @functools.partial(
        pl.kernel,
        out_type=[
            jax.ShapeDtypeStruct((BATCH,), jnp.int32),
            jax.ShapeDtypeStruct((BATCH,), jnp.int32),
            jax.ShapeDtypeStruct((BATCH * NUM_SURV,), jnp.int32),
            jax.ShapeDtypeStruct((BATCH * NUM_SURV,), jnp.int32),
        ],
        mesh=mesh,
        compiler_params=pltpu.CompilerParams(needs_layout_passes=False),
        scratch_types=[
            pltpu.VMEM((N_NODES,), jnp.int32),
            pltpu.VMEM((nb,), jnp.int32),
            pltpu.VMEM((nb,), jnp.int32),
            pltpu.VMEM((ns,), jnp.int32),
            pltpu.VMEM((ns,), jnp.int32),
            pltpu.VMEM((nb,), jnp.int32),
            pltpu.VMEM((nb,), jnp.int32),
            pltpu.VMEM((ns,), jnp.int32),
            pltpu.VMEM((ns,), jnp.int32),
            pltpu.SemaphoreType.DMA,
            pltpu.SemaphoreType.DMA,
        ],
    )
    def k(assoc_h, src_h, pos_h, negd_h, negs_h,
          oisrc_h, oidst_h, oind_h, oins_h,
          assoc_v, src_v, pos_v, negd_v, negs_v,
          isrc_v, idst_v, ind_v, ins_v, sem_in, sem_out):
        wid = lax.axis_index("s") * num_cores + lax.axis_index("c")
        copies = [
            pltpu.async_copy(src_h.at[pl.ds(wid * nb, nb)], src_v, sem_in),
            pltpu.async_copy(pos_h.at[pl.ds(wid * nb, nb)], pos_v, sem_in),
            pltpu.async_copy(negd_h.at[pl.ds(wid * ns, ns)], negd_v, sem_in),
            pltpu.async_copy(negs_h.at[pl.ds(wid * ns, ns)], negs_v, sem_in),
            pltpu.async_copy(assoc_h, assoc_v, sem_in),
        ]
        for c in copies:
            c.wait()

        def lam_body(t, carry):
            for u in range(2):
                sl = pl.ds(t * 32 + u * 16, 16)
                isrc_v[sl] = plsc.load_gather(assoc_v, [src_v[sl]])
                idst_v[sl] = plsc.load_gather(assoc_v, [pos_v[sl]])
            return carry

        lax.fori_loop(0, nb // 32, lam_body, 0)
        o1 = pltpu.async_copy(isrc_v, oisrc_h.at[pl.ds(wid * nb, nb)], sem_out)
        o2 = pltpu.async_copy(idst_v, oidst_h.at[pl.ds(wid * nb, nb)], sem_out)

        def surv_body(t, carry):
            for u in range(4):
                sl = pl.ds(t * 64 + u * 16, 16)
                ind_v[sl] = plsc.load_gather(assoc_v, [negd_v[sl]])
                ins_v[sl] = plsc.load_gather(assoc_v, [negs_v[sl]])
            return carry

        lax.fori_loop(0, ns // 64, surv_body, 0)
        o3 = pltpu.async_copy(ind_v, oind_h.at[pl.ds(wid * ns, ns)], sem_out)
        o4 = pltpu.async_copy(ins_v, oins_h.at[pl.ds(wid * ns, ns)], sem_out)
        o1.wait()
        o2.wait()
        o3.wait()
        o4.wait()

    return k(assoc, src, pos_dst, neg_dst_surv, neg_src_surv)


def _sc_logits(p, q, isrc, idst, ind, ins):
    mesh, nw, nb, ns = _sc_mesh()
    num_cores = mesh.num_cores

    @functools.partial(
        pl.kernel,
        out_type=[
            jax.ShapeDtypeStruct((BATCH,), jnp.float32),
            jax.ShapeDtypeStruct((BATCH * NUM_SURV,), jnp.float32),
            jax.ShapeDtypeStruct((BATCH * NUM_SURV,), jnp.float32),
        ],
        mesh=mesh,
        compiler_params=pltpu.CompilerParams(needs_layout_passes=False),
        scratch_types=[
            pltpu.VMEM((_N_PAD,), jnp.float32),
            pltpu.VMEM((_N_PAD,), jnp.float32),
            pltpu.VMEM((nb,), jnp.int32),
            pltpu.VMEM((nb,), jnp.int32),
            pltpu.VMEM((ns,), jnp.int32),
            pltpu.VMEM((ns,), jnp.int32),
            pltpu.VMEM((nb,), jnp.float32),
            pltpu.VMEM((nb,), jnp.float32),
            pltpu.VMEM((nb,), jnp.float32),
            pltpu.VMEM((ns,), jnp.float32),
            pltpu.VMEM((ns,), jnp.float32),
            pltpu.SemaphoreType.DMA,
            pltpu.SemaphoreType.DMA,
            pltpu.SemaphoreType.DMA,
        ],
    )
    def k(p_h, q_h, isrc_h, idst_h, ind_h, ins_h,
          olam_h, osu_h, osv_h,
          p_v, q_v, isrc_v, idst_v, ind_v, ins_v,
          psrc_v, qdst_v, lam_v, su_v, sv_v, sem_lam, sem_surv, sem_out):
        wid = lax.axis_index("s") * num_cores + lax.axis_index("c")
        lam_copies = [
            pltpu.async_copy(isrc_h.at[pl.ds(wid * nb, nb)], isrc_v, sem_lam),
            pltpu.async_copy(idst_h.at[pl.ds(wid * nb, nb)], idst_v, sem_lam),
            pltpu.async_copy(p_h, p_v, sem_lam),
            pltpu.async_copy(q_h, q_v, sem_lam),
        ]
        surv_copies = [
            pltpu.async_copy(ind_h.at[pl.ds(wid * ns, ns)], ind_v, sem_surv),
            pltpu.async_copy(ins_h.at[pl.ds(wid * ns, ns)], ins_v, sem_surv),
        ]
        for c in lam_copies:
            c.wait()

        def lam_body(t, carry):
            for u in range(2):
                sl = pl.ds(t * 32 + u * 16, 16)
                pe = plsc.load_gather(p_v, [isrc_v[sl]])
                qe = plsc.load_gather(q_v, [idst_v[sl]])
                psrc_v[sl] = pe
                qdst_v[sl] = qe
                lam_v[sl] = pe + qe
            return carry

        lax.fori_loop(0, nb // 32, lam_body, 0)
        out_lam = pltpu.async_copy(lam_v, olam_h.at[pl.ds(wid * nb, nb)], sem_out)
        for c in surv_copies:
            c.wait()

        lane = lax.iota(jnp.int32, 16)

        def surv_body(t, carry):
            for u in range(4):
                base = t * 64 + u * 16
                sl = pl.ds(base, 16)
                jv = lane + base
                ri = lax.shift_right_logical(jv * 6554, 16)
                pe = plsc.load_gather(psrc_v, [ri])
                qe = plsc.load_gather(q_v, [ind_v[sl]])
                su_v[sl] = pe + qe
                pe2 = plsc.load_gather(p_v, [ins_v[sl]])
                qe2 = plsc.load_gather(qdst_v, [ri])
                sv_v[sl] = pe2 + qe2
            return carry

        lax.fori_loop(0, ns // 64, surv_body, 0)

        out_su = pltpu.async_copy(su_v, osu_h.at[pl.ds(wid * ns, ns)], sem_out)
        out_sv = pltpu.async_copy(sv_v, osv_h.at[pl.ds(wid * ns, ns)], sem_out)
        out_lam.wait()
        out_su.wait()
        out_sv.wait()

    return k(p, q, isrc, idst, ind, ins)


def _loss_body(b_ref, psi_ref, slam_ref, ssu_ref, ssv_ref, o1, o2, o3):
    b = b_ref[0]
    psi = psi_ref[0]
    pe = psi + 1e-7
    gl = (slam_ref[...] + b) / pe
    lam = psi * (jnp.log(1.0 + jnp.exp(-gl)) + gl)
    o1[0] = -jnp.sum(jnp.log(lam + 1e-10)) / BATCH
    gu = (ssu_ref[...] + b) / pe
    o2[0] = psi * jnp.sum(jnp.log(1.0 + jnp.exp(-gu)) + gu) / NUM_SURV / BATCH
    gv = (ssv_ref[...] + b) / pe
    o3[0] = psi * jnp.sum(jnp.log(1.0 + jnp.exp(-gv)) + gv) / NUM_SURV / BATCH


def _losses(s_lam, s_su, s_sv, b_omega, psi):
    o1, o2, o3 = pl.pallas_call(
        _loss_body,
        in_specs=[
            pl.BlockSpec(memory_space=pltpu.SMEM),
            pl.BlockSpec(memory_space=pltpu.SMEM),
            pl.BlockSpec((BATCH // 128, 128), lambda: (0, 0)),
            pl.BlockSpec((BATCH * NUM_SURV // 128, 128), lambda: (0, 0)),
            pl.BlockSpec((BATCH * NUM_SURV // 128, 128), lambda: (0, 0)),
        ],
        out_specs=[
            pl.BlockSpec(memory_space=pltpu.SMEM),
            pl.BlockSpec(memory_space=pltpu.SMEM),
            pl.BlockSpec(memory_space=pltpu.SMEM),
        ],
        out_shape=[jax.ShapeDtypeStruct((1,), jnp.float32)] * 3,
    )(
        b_omega, psi,
        s_lam.reshape(BATCH // 128, 128),
        s_su.reshape(BATCH * NUM_SURV // 128, 128),
        s_sv.reshape(BATCH * NUM_SURV // 128, 128),
    )
    return o1[0], o2[0], o3[0]


def kernel(all_embeddings, assoc, src, pos_dst, neg_dst_surv, neg_src_surv,
           W_omega, b_omega, psi):
    assoc_i = assoc.astype(jnp.int32)
    isrc, idst, ind, ins = _sc_compose(
        assoc_i, src, pos_dst, neg_dst_surv, neg_src_surv)
    p, q = _compute_pq(all_embeddings, W_omega)
    s_lam, s_su, s_sv = _sc_logits(p, q, isrc, idst, ind, ins)
    return _losses(s_lam, s_su, s_sv, b_omega, psi)
